# Initial kernel scaffold; baseline (speedup 1.0000x reference)
import jax
import jax.numpy as jnp
from jax.experimental import pallas as pl


def kernel(x_reviewer, x_author, edge_index_r2a, edge_index_a2r, W_proj_rev, b_proj_rev, W_proj_aut, b_proj_aut, a_src_r2a, a_dst_r2a, a_src_a2r, a_dst_a2r, q_sem, W_k, b_k, W_fc, b_fc):
    raise NotImplementedError("write your pallas kernel here")


# speedup vs baseline: 6.2436x; 6.2436x over previous
import functools

import jax
import jax.numpy as jnp
from jax import lax
from jax.experimental import pallas as pl
from jax.experimental.pallas import tpu as pltpu
from jax.experimental.pallas import tpu_sc as plsc

N = 50000
NP = 50048
E = 500000
D_IN = 128
HID = 128
H = 8
DH = 16
OUT = 64
NC = 2
NS = 16
NW = NC * NS
EB = 128
NBLK = 123
PER_SUB = NBLK * EB
EP = PER_SUB * NW
EPH = EP // 2
ROWS_PER_SUB = NP // NS


def _proj_body(x_ref, w_ref, b_ref, a1_ref, a2_ref, xp_ref, al1_ref, al2_ref):
    xp = jnp.dot(x_ref[...], w_ref[...], preferred_element_type=jnp.float32)
    xp = xp + b_ref[...]
    xp_ref[...] = xp
    al1_ref[...] = jnp.dot(xp, a1_ref[...], preferred_element_type=jnp.float32)
    al2_ref[...] = jnp.dot(xp, a2_ref[...], preferred_element_type=jnp.float32)


def _proj(x, W, b, A1, A2):
    RB = 1000
    nb = N // RB
    return pl.pallas_call(
        _proj_body,
        grid=(nb,),
        in_specs=[
            pl.BlockSpec((RB, D_IN), lambda i: (i, 0)),
            pl.BlockSpec((D_IN, HID), lambda i: (0, 0)),
            pl.BlockSpec((1, HID), lambda i: (0, 0)),
            pl.BlockSpec((HID, 16), lambda i: (0, 0)),
            pl.BlockSpec((HID, 16), lambda i: (0, 0)),
        ],
        out_specs=[
            pl.BlockSpec((RB, HID), lambda i: (i, 0)),
            pl.BlockSpec((RB, 16), lambda i: (i, 0)),
            pl.BlockSpec((RB, 16), lambda i: (i, 0)),
        ],
        out_shape=[
            jax.ShapeDtypeStruct((N, HID), jnp.float32),
            jax.ShapeDtypeStruct((N, 16), jnp.float32),
            jax.ShapeDtypeStruct((N, 16), jnp.float32),
        ],
    )(x, W, b.reshape(1, HID), A1, A2)


_MESH = plsc.VectorSubcoreMesh(core_axis_name="c", subcore_axis_name="s")
_SC_PARAMS = pltpu.CompilerParams(use_tc_tiling_on_sc=False)


def _m8(x):
    return pl.multiple_of(x, 8)


@functools.partial(
    pl.kernel, mesh=_MESH, compiler_params=_SC_PARAMS,
    out_type=[
        jax.ShapeDtypeStruct((EP, 16), jnp.float32),
        jax.ShapeDtypeStruct((NP, 16), jnp.float32),
        jax.ShapeDtypeStruct((NP, 16), jnp.float32),
    ],
    scratch_types=[
        pltpu.VMEM((EB,), jnp.int32),
        pltpu.VMEM((EB,), jnp.int32),
        pltpu.VMEM((EB, 16), jnp.float32),
        pltpu.VMEM((EB, 16), jnp.float32),
        pltpu.VMEM((EB, 16), jnp.float32),
        pltpu.VMEM((ROWS_PER_SUB, 16), jnp.float32),
        pltpu.VMEM_SHARED((NP, 16), jnp.float32),
    ],
)
def _k1(als_hbm, ald_hbm, src_hbm, dst_hbm,
        ex_hbm, den0_hbm, den1_hbm,
        sidx, didx, as_v, ad_v, exf, zbuf, den_sh):
    cid = lax.axis_index("c")
    sid = lax.axis_index("s")
    wid = sid * NC + cid
    zeros16 = jnp.zeros((16,), jnp.float32)

    def zb(i, _):
        zbuf[i, :] = zeros16
        return 0
    lax.fori_loop(0, ROWS_PER_SUB, zb, 0)
    pltpu.sync_copy(zbuf, den_sh.at[pl.ds(_m8(sid * ROWS_PER_SUB), ROWS_PER_SUB)])
    plsc.subcore_barrier()

    base0 = wid * PER_SUB

    def blk(b, _):
        base = base0 + b * EB
        pltpu.sync_copy(src_hbm.at[pl.ds(_m8(base), EB)], sidx)
        pltpu.sync_copy(dst_hbm.at[pl.ds(_m8(base), EB)], didx)
        pltpu.sync_copy(als_hbm.at[sidx], as_v)
        pltpu.sync_copy(ald_hbm.at[didx], ad_v)

        def edge(i, _):
            s = as_v[i, :] + ad_v[i, :]
            s = jnp.maximum(s, 0.0) + 0.2 * jnp.minimum(s, 0.0)
            exf[i, :] = jnp.exp(s)
            return 0
        lax.fori_loop(0, EB, edge, 0)

        pltpu.sync_copy(exf, ex_hbm.at[pl.ds(_m8(base), EB)])
        pltpu.sync_copy(exf, den_sh.at[didx], add=True)
        return 0
    lax.fori_loop(0, NBLK, blk, 0)

    plsc.subcore_barrier()
    r0 = sid * ROWS_PER_SUB

    @pl.when(cid == 0)
    def _():
        pltpu.sync_copy(den_sh.at[pl.ds(_m8(r0), ROWS_PER_SUB)],
                        den0_hbm.at[pl.ds(_m8(r0), ROWS_PER_SUB)])

    @pl.when(cid == 1)
    def _():
        pltpu.sync_copy(den_sh.at[pl.ds(_m8(r0), ROWS_PER_SUB)],
                        den1_hbm.at[pl.ds(_m8(r0), ROWS_PER_SUB)])


@functools.partial(
    pl.kernel, mesh=_MESH, compiler_params=_SC_PARAMS,
    out_type=[jax.ShapeDtypeStruct((EP, 16), jnp.float32)],
    scratch_types=[
        pltpu.VMEM((EB,), jnp.int32),
        pltpu.VMEM((EB, 16), jnp.float32),
        pltpu.VMEM((EB, 16), jnp.float32),
        pltpu.VMEM((EB, 16), jnp.float32),
        pltpu.VMEM((EB, 16), jnp.float32),
    ],
)
def _k2(ex_hbm, den0_hbm, den1_hbm, dst_hbm, w_hbm,
        didx, exv, d0v, d1v, wv):
    cid = lax.axis_index("c")
    sid = lax.axis_index("s")
    wid = sid * NC + cid
    base0 = wid * PER_SUB

    def blk(b, _):
        base = base0 + b * EB
        pltpu.sync_copy(dst_hbm.at[pl.ds(_m8(base), EB)], didx)
        pltpu.sync_copy(ex_hbm.at[pl.ds(_m8(base), EB)], exv)
        pltpu.sync_copy(den0_hbm.at[didx], d0v)
        pltpu.sync_copy(den1_hbm.at[didx], d1v)

        def edge(i, _):
            wv[i, :] = exv[i, :] / (d0v[i, :] + d1v[i, :])
            return 0
        lax.fori_loop(0, EB, edge, 0)

        pltpu.sync_copy(wv, w_hbm.at[pl.ds(_m8(base), EB)])
        return 0
    lax.fori_loop(0, NBLK, blk, 0)


@functools.partial(
    pl.kernel, mesh=_MESH, compiler_params=_SC_PARAMS,
    out_type=[jax.ShapeDtypeStruct((8, NC, NP, 16), jnp.float32)],
    scratch_types=[
        pltpu.VMEM((EB,), jnp.int32),
        pltpu.VMEM((EB,), jnp.int32),
        pltpu.VMEM((EB, 16), jnp.float32),
        pltpu.VMEM((EB,), jnp.float32),
        pltpu.VMEM((ROWS_PER_SUB, 16), jnp.float32),
        pltpu.VMEM_SHARED((NP, 16), jnp.float32),
    ],
)
def _k3(xc0, xc1, xc2, xc3, xc4, xc5, xc6, xc7, wt_hbm, src_hbm, dst_hbm,
        acc_hbm, sidx, didx, rows, wrow, zbuf, acc_sh):
    cid = lax.axis_index("c")
    sid = lax.axis_index("s")
    wid = sid * NC + cid
    zeros16 = jnp.zeros((16,), jnp.float32)
    base0 = wid * PER_SUB
    r0 = sid * ROWS_PER_SUB

    def zb(i, _):
        zbuf[i, :] = zeros16
        return 0
    lax.fori_loop(0, ROWS_PER_SUB, zb, 0)

    for c in range(8):
        xc = (xc0, xc1, xc2, xc3, xc4, xc5, xc6, xc7)[c]
        pltpu.sync_copy(zbuf, acc_sh.at[pl.ds(_m8(r0), ROWS_PER_SUB)])
        plsc.subcore_barrier()

        def blk(b, _):
            base = base0 + b * EB
            pltpu.sync_copy(src_hbm.at[pl.ds(_m8(base), EB)], sidx)
            pltpu.sync_copy(dst_hbm.at[pl.ds(_m8(base), EB)], didx)
            pltpu.sync_copy(xc.at[sidx], rows)
            pltpu.sync_copy(wt_hbm.at[c, pl.ds(_m8(base), EB)], wrow)

            def grp(g, _):
                wvec = wrow[pl.ds(g * 16, 16)]
                for j in range(16):
                    i = g * 16 + j
                    gj = jnp.broadcast_to(wvec[j], (16,))
                    rows[i, :] = rows[i, :] * gj
                return 0
            lax.fori_loop(0, EB // 16, grp, 0)

            pltpu.sync_copy(rows, acc_sh.at[didx], add=True)
            return 0
        lax.fori_loop(0, NBLK, blk, 0)

        plsc.subcore_barrier()
        pltpu.sync_copy(acc_sh.at[pl.ds(_m8(r0), ROWS_PER_SUB)],
                        acc_hbm.at[c, cid, pl.ds(_m8(r0), ROWS_PER_SUB)])
        plsc.subcore_barrier()


def _wt_body(w_ref, out_ref):
    out_ref[...] = jnp.transpose(w_ref[...], (1, 0))[:8, :]


def _wt(w):
    BR = 4096
    nb = EP // BR
    return pl.pallas_call(
        _wt_body,
        grid=(nb,),
        in_specs=[pl.BlockSpec((BR, 16), lambda i: (i, 0))],
        out_specs=pl.BlockSpec((8, BR), lambda i: (0, i)),
        out_shape=jax.ShapeDtypeStruct((8, EP), jnp.float32),
    )(w)


def _final_body(acc_ref, wfc_ref, bfc_ref, out_ref):
    s = acc_ref[:, 0] + acc_ref[:, 1]
    feat = jnp.concatenate([s[c] for c in range(8)], axis=1)
    feat = jnp.maximum(feat, 0.0)
    out_ref[...] = jnp.dot(feat, wfc_ref[...],
                           preferred_element_type=jnp.float32) + bfc_ref[...]


def _final(acc, W_fc, b_fc):
    RB = 1000
    nb = N // RB
    return pl.pallas_call(
        _final_body,
        grid=(nb,),
        in_specs=[
            pl.BlockSpec((8, NC, RB, 16), lambda i: (0, 0, i, 0)),
            pl.BlockSpec((HID, OUT), lambda i: (0, 0)),
            pl.BlockSpec((1, OUT), lambda i: (0, 0)),
        ],
        out_specs=pl.BlockSpec((RB, OUT), lambda i: (i, 0)),
        out_shape=jax.ShapeDtypeStruct((N, OUT), jnp.float32),
    )(acc, W_fc, b_fc.reshape(1, OUT))


def _blockdiag(a):
    eye = jnp.eye(H, 16, dtype=jnp.float32)
    return (a[:, :, None] * eye[:, None, :]).reshape(HID, 16)


def _pad_edges(ei):
    src = jnp.pad(ei[0], (0, EP - E), constant_values=0)
    dst = jnp.pad(ei[1], (0, EP - E), constant_values=N)
    return src.astype(jnp.int32), dst.astype(jnp.int32)


def _edge_type(als, ald_padded, xc, src, dst):
    ex, d0, d1 = _k1(als, ald_padded, src, dst)
    (w,) = _k2(ex, d0, d1, dst)
    wt = _wt(w)
    (acc,) = _k3(*xc, wt, src, dst)
    return acc


def kernel(x_reviewer, x_author, edge_index_r2a, edge_index_a2r,
           W_proj_rev, b_proj_rev, W_proj_aut, b_proj_aut,
           a_src_r2a, a_dst_r2a, a_src_a2r, a_dst_a2r,
           q_sem, W_k, b_k, W_fc, b_fc):
    xpR, alR_s_r2a, alR_d_a2r = _proj(
        x_reviewer, W_proj_rev, b_proj_rev,
        _blockdiag(a_src_r2a), _blockdiag(a_dst_a2r))
    xpA, alA_s_a2r, alA_d_r2a = _proj(
        x_author, W_proj_aut, b_proj_aut,
        _blockdiag(a_src_a2r), _blockdiag(a_dst_r2a))

    xcR = [xpR[:, 16 * c:16 * c + 16] for c in range(8)]
    xcA = [xpA[:, 16 * c:16 * c + 16] for c in range(8)]
    padN = ((0, NP - N), (0, 0))

    srcA, dstA = _pad_edges(edge_index_r2a)
    srcB, dstB = _pad_edges(edge_index_a2r)

    acc_aut = _edge_type(alR_s_r2a, jnp.pad(alA_d_r2a, padN), xcR, srcA, dstA)
    acc_rev = _edge_type(alA_s_a2r, jnp.pad(alR_d_a2r, padN), xcA, srcB, dstB)

    out_rev = _final(acc_rev, W_fc, b_fc)
    out_aut = _final(acc_aut, W_fc, b_fc)
    return jnp.concatenate([out_rev, out_aut], axis=0)


# speedup vs baseline: 8.4601x; 1.3550x over previous
import functools

import jax
import jax.numpy as jnp
from jax import lax
from jax.experimental import pallas as pl
from jax.experimental.pallas import tpu as pltpu
from jax.experimental.pallas import tpu_sc as plsc

N = 50000
NP = 50048
E = 500000
D_IN = 128
HID = 128
H = 8
DH = 16
OUT = 64
NC = 2
NS = 16
NW = NC * NS
EB = 512
NBLK = 32
PER_SUB = NBLK * EB
EP = PER_SUB * NW
ER = EP // 128
ROWS_PER_SUB = NP // NS


def _proj_body(x_ref, w_ref, b_ref, a1_ref, a2_ref, xp_ref, al1_ref, al2_ref):
    xp = jnp.dot(x_ref[...], w_ref[...], preferred_element_type=jnp.float32)
    xp = xp + b_ref[...]
    xp_ref[...] = xp
    al1_ref[...] = jnp.dot(xp, a1_ref[...], preferred_element_type=jnp.float32)
    al2_ref[...] = jnp.dot(xp, a2_ref[...], preferred_element_type=jnp.float32)


def _proj(x, W, b, A1, A2):
    RB = 1000
    nb = N // RB
    return pl.pallas_call(
        _proj_body,
        grid=(nb,),
        in_specs=[
            pl.BlockSpec((RB, D_IN), lambda i: (i, 0)),
            pl.BlockSpec((D_IN, HID), lambda i: (0, 0)),
            pl.BlockSpec((1, HID), lambda i: (0, 0)),
            pl.BlockSpec((HID, 16), lambda i: (0, 0)),
            pl.BlockSpec((HID, 16), lambda i: (0, 0)),
        ],
        out_specs=[
            pl.BlockSpec((RB, HID), lambda i: (i, 0)),
            pl.BlockSpec((RB, 16), lambda i: (i, 0)),
            pl.BlockSpec((RB, 16), lambda i: (i, 0)),
        ],
        out_shape=[
            jax.ShapeDtypeStruct((N, HID), jnp.float32),
            jax.ShapeDtypeStruct((N, 16), jnp.float32),
            jax.ShapeDtypeStruct((N, 16), jnp.float32),
        ],
    )(x, W, b.reshape(1, HID), A1, A2)


_MESH = plsc.VectorSubcoreMesh(core_axis_name="c", subcore_axis_name="s")
_SC_PARAMS = pltpu.CompilerParams(use_tc_tiling_on_sc=False)


def _m8(x):
    return pl.multiple_of(x, 8)


def _m4(x):
    return pl.multiple_of(x, 4)


@functools.partial(
    pl.kernel, mesh=_MESH, compiler_params=_SC_PARAMS,
    out_type=[
        jax.ShapeDtypeStruct((EP, 16), jnp.float32),
        jax.ShapeDtypeStruct((NP, 16), jnp.float32),
        jax.ShapeDtypeStruct((NP, 16), jnp.float32),
    ],
    scratch_types=[
        pltpu.VMEM((4, 128), jnp.int32),
        pltpu.VMEM((4, 128), jnp.int32),
        pltpu.VMEM((EB, 16), jnp.float32),
        pltpu.VMEM((EB, 16), jnp.float32),
        pltpu.VMEM((EB, 16), jnp.float32),
        pltpu.VMEM((ROWS_PER_SUB, 16), jnp.float32),
        pltpu.VMEM_SHARED((NP, 16), jnp.float32),
        pltpu.SemaphoreType.DMA,
    ],
)
def _k1(als_hbm, ald_hbm, src_hbm, dst_hbm,
        ex_hbm, den0_hbm, den1_hbm,
        sidx, didx, as_v, ad_v, exf, zbuf, den_sh, sem):
    cid = lax.axis_index("c")
    sid = lax.axis_index("s")
    wid = sid * NC + cid
    zeros16 = jnp.zeros((16,), jnp.float32)

    def zb(i, _):
        zbuf[i, :] = zeros16
        return 0
    lax.fori_loop(0, ROWS_PER_SUB, zb, 0)
    pltpu.sync_copy(zbuf, den_sh.at[pl.ds(_m8(sid * ROWS_PER_SUB), ROWS_PER_SUB)])
    plsc.subcore_barrier()

    base0 = wid * PER_SUB
    row0_0 = wid * NBLK * 4

    def blk(b, _):
        base = base0 + b * EB
        row0 = row0_0 + b * 4
        pltpu.sync_copy(src_hbm.at[pl.ds(_m4(row0), 4)], sidx)
        pltpu.sync_copy(dst_hbm.at[pl.ds(_m4(row0), 4)], didx)
        cps = []
        for t in range(4):
            cps.append(pltpu.async_copy(
                als_hbm.at[sidx.at[t]], as_v.at[pl.ds(128 * t, 128)], sem))
            cps.append(pltpu.async_copy(
                ald_hbm.at[didx.at[t]], ad_v.at[pl.ds(128 * t, 128)], sem))
        for cp in cps:
            cp.wait()

        def edge(i, _):
            s = as_v[i, :] + ad_v[i, :]
            s = jnp.maximum(s, 0.0) + 0.2 * jnp.minimum(s, 0.0)
            exf[i, :] = jnp.exp(s)
            return 0
        lax.fori_loop(0, EB, edge, 0)

        pltpu.sync_copy(exf, ex_hbm.at[pl.ds(_m8(base), EB)])
        for t in range(4):
            pltpu.sync_copy(exf.at[pl.ds(128 * t, 128)],
                            den_sh.at[didx.at[t]], add=True)
        return 0
    lax.fori_loop(0, NBLK, blk, 0)

    plsc.subcore_barrier()
    r0 = sid * ROWS_PER_SUB

    @pl.when(cid == 0)
    def _():
        pltpu.sync_copy(den_sh.at[pl.ds(_m8(r0), ROWS_PER_SUB)],
                        den0_hbm.at[pl.ds(_m8(r0), ROWS_PER_SUB)])

    @pl.when(cid == 1)
    def _():
        pltpu.sync_copy(den_sh.at[pl.ds(_m8(r0), ROWS_PER_SUB)],
                        den1_hbm.at[pl.ds(_m8(r0), ROWS_PER_SUB)])


@functools.partial(
    pl.kernel, mesh=_MESH, compiler_params=_SC_PARAMS,
    out_type=[jax.ShapeDtypeStruct((EP, 16), jnp.float32)],
    scratch_types=[
        pltpu.VMEM((4, 128), jnp.int32),
        pltpu.VMEM((EB, 16), jnp.float32),
        pltpu.VMEM((EB, 16), jnp.float32),
        pltpu.VMEM((EB, 16), jnp.float32),
        pltpu.VMEM((EB, 16), jnp.float32),
        pltpu.SemaphoreType.DMA,
    ],
)
def _k2(ex_hbm, den0_hbm, den1_hbm, dst_hbm, w_hbm,
        didx, exv, d0v, d1v, wv, sem):
    cid = lax.axis_index("c")
    sid = lax.axis_index("s")
    wid = sid * NC + cid
    base0 = wid * PER_SUB
    row0_0 = wid * NBLK * 4

    def blk(b, _):
        base = base0 + b * EB
        row0 = row0_0 + b * 4
        pltpu.sync_copy(dst_hbm.at[pl.ds(_m4(row0), 4)], didx)
        cps = [pltpu.async_copy(ex_hbm.at[pl.ds(_m8(base), EB)], exv, sem)]
        for t in range(4):
            cps.append(pltpu.async_copy(
                den0_hbm.at[didx.at[t]], d0v.at[pl.ds(128 * t, 128)], sem))
            cps.append(pltpu.async_copy(
                den1_hbm.at[didx.at[t]], d1v.at[pl.ds(128 * t, 128)], sem))
        for cp in cps:
            cp.wait()

        def edge(i, _):
            wv[i, :] = exv[i, :] / (d0v[i, :] + d1v[i, :])
            return 0
        lax.fori_loop(0, EB, edge, 0)

        pltpu.sync_copy(wv, w_hbm.at[pl.ds(_m8(base), EB)])
        return 0
    lax.fori_loop(0, NBLK, blk, 0)


@functools.partial(
    pl.kernel, mesh=_MESH, compiler_params=_SC_PARAMS,
    out_type=[jax.ShapeDtypeStruct((8, NC, NP, 16), jnp.float32)],
    scratch_types=[
        pltpu.VMEM((4, 128), jnp.int32),
        pltpu.VMEM((4, 128), jnp.int32),
        pltpu.VMEM((EB, 16), jnp.float32),
        pltpu.VMEM((EB,), jnp.float32),
        pltpu.VMEM((ROWS_PER_SUB, 16), jnp.float32),
        pltpu.VMEM_SHARED((NP, 16), jnp.float32),
        pltpu.SemaphoreType.DMA,
    ],
)
def _k3(xc0, xc1, xc2, xc3, xc4, xc5, xc6, xc7, wt_hbm, src_hbm, dst_hbm,
        acc_hbm, sidx, didx, rows, wrow, zbuf, acc_sh, sem):
    cid = lax.axis_index("c")
    sid = lax.axis_index("s")
    wid = sid * NC + cid
    zeros16 = jnp.zeros((16,), jnp.float32)
    base0 = wid * PER_SUB
    row0_0 = wid * NBLK * 4
    r0 = sid * ROWS_PER_SUB

    def zb(i, _):
        zbuf[i, :] = zeros16
        return 0
    lax.fori_loop(0, ROWS_PER_SUB, zb, 0)

    for c in range(8):
        xc = (xc0, xc1, xc2, xc3, xc4, xc5, xc6, xc7)[c]
        pltpu.sync_copy(zbuf, acc_sh.at[pl.ds(_m8(r0), ROWS_PER_SUB)])
        plsc.subcore_barrier()

        def blk(b, _):
            base = base0 + b * EB
            row0 = row0_0 + b * 4
            pltpu.sync_copy(src_hbm.at[pl.ds(_m4(row0), 4)], sidx)
            pltpu.sync_copy(dst_hbm.at[pl.ds(_m4(row0), 4)], didx)
            cps = [pltpu.async_copy(
                wt_hbm.at[c, pl.ds(_m8(base), EB)], wrow, sem)]
            for t in range(4):
                cps.append(pltpu.async_copy(
                    xc.at[sidx.at[t]], rows.at[pl.ds(128 * t, 128)], sem))
            for cp in cps:
                cp.wait()

            def grp(g, _):
                wvec = wrow[pl.ds(g * 16, 16)]
                for j in range(16):
                    i = g * 16 + j
                    gj = jnp.broadcast_to(wvec[j], (16,))
                    rows[i, :] = rows[i, :] * gj
                return 0
            lax.fori_loop(0, EB // 16, grp, 0)

            for t in range(4):
                pltpu.sync_copy(rows.at[pl.ds(128 * t, 128)],
                                acc_sh.at[didx.at[t]], add=True)
            return 0
        lax.fori_loop(0, NBLK, blk, 0)

        plsc.subcore_barrier()
        pltpu.sync_copy(acc_sh.at[pl.ds(_m8(r0), ROWS_PER_SUB)],
                        acc_hbm.at[c, cid, pl.ds(_m8(r0), ROWS_PER_SUB)])
        plsc.subcore_barrier()


def _wt_body(w_ref, out_ref):
    out_ref[...] = jnp.transpose(w_ref[...], (1, 0))[:8, :]


def _wt(w):
    BR = 4096
    nb = EP // BR
    return pl.pallas_call(
        _wt_body,
        grid=(nb,),
        in_specs=[pl.BlockSpec((BR, 16), lambda i: (i, 0))],
        out_specs=pl.BlockSpec((8, BR), lambda i: (0, i)),
        out_shape=jax.ShapeDtypeStruct((8, EP), jnp.float32),
    )(w)


def _final_body(acc_ref, wfc_ref, bfc_ref, out_ref):
    s = acc_ref[:, 0] + acc_ref[:, 1]
    feat = jnp.concatenate([s[c] for c in range(8)], axis=1)
    feat = jnp.maximum(feat, 0.0)
    out_ref[...] = jnp.dot(feat, wfc_ref[...],
                           preferred_element_type=jnp.float32) + bfc_ref[...]


def _final(acc, W_fc, b_fc):
    RB = 1000
    nb = N // RB
    return pl.pallas_call(
        _final_body,
        grid=(nb,),
        in_specs=[
            pl.BlockSpec((8, NC, RB, 16), lambda i: (0, 0, i, 0)),
            pl.BlockSpec((HID, OUT), lambda i: (0, 0)),
            pl.BlockSpec((1, OUT), lambda i: (0, 0)),
        ],
        out_specs=pl.BlockSpec((RB, OUT), lambda i: (i, 0)),
        out_shape=jax.ShapeDtypeStruct((N, OUT), jnp.float32),
    )(acc, W_fc, b_fc.reshape(1, OUT))


def _blockdiag(a):
    eye = jnp.eye(H, 16, dtype=jnp.float32)
    return (a[:, :, None] * eye[:, None, :]).reshape(HID, 16)


def _pad_edges(ei):
    src = jnp.pad(ei[0], (0, EP - E), constant_values=0)
    dst = jnp.pad(ei[1], (0, EP - E), constant_values=N)
    return (src.astype(jnp.int32).reshape(ER, 128),
            dst.astype(jnp.int32).reshape(ER, 128))


def _edge_type(als, ald_padded, xc, src, dst):
    ex, d0, d1 = _k1(als, ald_padded, src, dst)
    (w,) = _k2(ex, d0, d1, dst)
    wt = _wt(w)
    (acc,) = _k3(*xc, wt, src, dst)
    return acc


def kernel(x_reviewer, x_author, edge_index_r2a, edge_index_a2r,
           W_proj_rev, b_proj_rev, W_proj_aut, b_proj_aut,
           a_src_r2a, a_dst_r2a, a_src_a2r, a_dst_a2r,
           q_sem, W_k, b_k, W_fc, b_fc):
    xpR, alR_s_r2a, alR_d_a2r = _proj(
        x_reviewer, W_proj_rev, b_proj_rev,
        _blockdiag(a_src_r2a), _blockdiag(a_dst_a2r))
    xpA, alA_s_a2r, alA_d_r2a = _proj(
        x_author, W_proj_aut, b_proj_aut,
        _blockdiag(a_src_a2r), _blockdiag(a_dst_r2a))

    xcR = [xpR[:, 16 * c:16 * c + 16] for c in range(8)]
    xcA = [xpA[:, 16 * c:16 * c + 16] for c in range(8)]
    padN = ((0, NP - N), (0, 0))

    srcA, dstA = _pad_edges(edge_index_r2a)
    srcB, dstB = _pad_edges(edge_index_a2r)

    acc_aut = _edge_type(alR_s_r2a, jnp.pad(alA_d_r2a, padN), xcR, srcA, dstA)
    acc_rev = _edge_type(alA_s_a2r, jnp.pad(alR_d_a2r, padN), xcA, srcB, dstB)

    out_rev = _final(acc_rev, W_fc, b_fc)
    out_aut = _final(acc_aut, W_fc, b_fc)
    return jnp.concatenate([out_rev, out_aut], axis=0)


# speedup vs baseline: 10.0862x; 1.1922x over previous
import functools

import jax
import jax.numpy as jnp
from jax import lax
from jax.experimental import pallas as pl
from jax.experimental.pallas import tpu as pltpu
from jax.experimental.pallas import tpu_sc as plsc

N = 50000
NP = 50048
E = 500000
D_IN = 128
HID = 128
H = 8
DH = 16
OUT = 64
NC = 2
NS = 16
NW = NC * NS
EB = 512
NBLK = 32
PER_SUB = NBLK * EB
EP = PER_SUB * NW
ER = EP // 128
ROWS_PER_SUB = NP // NS


def _proj_body(x_ref, w_ref, b_ref, a1_ref, a2_ref, xp_ref, al1_ref, al2_ref):
    xp = jnp.dot(x_ref[...], w_ref[...], preferred_element_type=jnp.float32)
    xp = xp + b_ref[...]
    xp_ref[...] = xp
    al1_ref[...] = jnp.dot(xp, a1_ref[...], preferred_element_type=jnp.float32)
    al2_ref[...] = jnp.dot(xp, a2_ref[...], preferred_element_type=jnp.float32)


def _proj(x, W, b, A1, A2):
    RB = 1000
    nb = N // RB
    return pl.pallas_call(
        _proj_body,
        grid=(nb,),
        in_specs=[
            pl.BlockSpec((RB, D_IN), lambda i: (i, 0)),
            pl.BlockSpec((D_IN, HID), lambda i: (0, 0)),
            pl.BlockSpec((1, HID), lambda i: (0, 0)),
            pl.BlockSpec((HID, 16), lambda i: (0, 0)),
            pl.BlockSpec((HID, 16), lambda i: (0, 0)),
        ],
        out_specs=[
            pl.BlockSpec((RB, HID), lambda i: (i, 0)),
            pl.BlockSpec((RB, 16), lambda i: (i, 0)),
            pl.BlockSpec((RB, 16), lambda i: (i, 0)),
        ],
        out_shape=[
            jax.ShapeDtypeStruct((N, HID), jnp.float32),
            jax.ShapeDtypeStruct((N, 16), jnp.float32),
            jax.ShapeDtypeStruct((N, 16), jnp.float32),
        ],
    )(x, W, b.reshape(1, HID), A1, A2)


_MESH = plsc.VectorSubcoreMesh(core_axis_name="c", subcore_axis_name="s")
_SC_PARAMS = pltpu.CompilerParams(use_tc_tiling_on_sc=False)


def _m8(x):
    return pl.multiple_of(x, 8)


def _m4(x):
    return pl.multiple_of(x, 4)


@functools.partial(
    pl.kernel, mesh=_MESH, compiler_params=_SC_PARAMS,
    out_type=[
        jax.ShapeDtypeStruct((EP, 16), jnp.float32),
        jax.ShapeDtypeStruct((NP, 16), jnp.float32),
        jax.ShapeDtypeStruct((NP, 16), jnp.float32),
    ],
    scratch_types=[
        pltpu.VMEM((4, 128), jnp.int32),
        pltpu.VMEM((4, 128), jnp.int32),
        pltpu.VMEM((EB, 16), jnp.float32),
        pltpu.VMEM((EB, 16), jnp.float32),
        pltpu.VMEM((EB, 16), jnp.float32),
        pltpu.VMEM((ROWS_PER_SUB, 16), jnp.float32),
        pltpu.VMEM_SHARED((NP, 16), jnp.float32),
        pltpu.SemaphoreType.DMA,
    ],
)
def _k1(als_hbm, ald_hbm, src_hbm, dst_hbm,
        ex_hbm, den0_hbm, den1_hbm,
        sidx, didx, as_v, ad_v, exf, zbuf, den_sh, sem):
    cid = lax.axis_index("c")
    sid = lax.axis_index("s")
    wid = sid * NC + cid
    zeros16 = jnp.zeros((16,), jnp.float32)

    def zb(i, _):
        zbuf[i, :] = zeros16
        return 0
    lax.fori_loop(0, ROWS_PER_SUB, zb, 0)
    pltpu.sync_copy(zbuf, den_sh.at[pl.ds(_m8(sid * ROWS_PER_SUB), ROWS_PER_SUB)])
    plsc.subcore_barrier()

    base0 = wid * PER_SUB
    row0_0 = wid * NBLK * 4

    def blk(b, _):
        base = base0 + b * EB
        row0 = row0_0 + b * 4
        pltpu.sync_copy(src_hbm.at[pl.ds(_m4(row0), 4)], sidx)
        pltpu.sync_copy(dst_hbm.at[pl.ds(_m4(row0), 4)], didx)
        cps = []
        for t in range(4):
            cps.append(pltpu.async_copy(
                als_hbm.at[sidx.at[t]], as_v.at[pl.ds(128 * t, 128)], sem))
            cps.append(pltpu.async_copy(
                ald_hbm.at[didx.at[t]], ad_v.at[pl.ds(128 * t, 128)], sem))
        for cp in cps:
            cp.wait()

        def edge(i, _):
            s = as_v[i, :] + ad_v[i, :]
            s = jnp.maximum(s, 0.0) + 0.2 * jnp.minimum(s, 0.0)
            exf[i, :] = jnp.exp(s)
            return 0
        lax.fori_loop(0, EB, edge, 0)

        pltpu.sync_copy(exf, ex_hbm.at[pl.ds(_m8(base), EB)])
        for t in range(4):
            pltpu.sync_copy(exf.at[pl.ds(128 * t, 128)],
                            den_sh.at[didx.at[t]], add=True)
        return 0
    lax.fori_loop(0, NBLK, blk, 0)

    plsc.subcore_barrier()
    r0 = sid * ROWS_PER_SUB

    @pl.when(cid == 0)
    def _():
        pltpu.sync_copy(den_sh.at[pl.ds(_m8(r0), ROWS_PER_SUB)],
                        den0_hbm.at[pl.ds(_m8(r0), ROWS_PER_SUB)])

    @pl.when(cid == 1)
    def _():
        pltpu.sync_copy(den_sh.at[pl.ds(_m8(r0), ROWS_PER_SUB)],
                        den1_hbm.at[pl.ds(_m8(r0), ROWS_PER_SUB)])


@functools.partial(
    pl.kernel, mesh=_MESH, compiler_params=_SC_PARAMS,
    out_type=[jax.ShapeDtypeStruct((EP, 16), jnp.float32)],
    scratch_types=[
        pltpu.VMEM((4, 128), jnp.int32),
        pltpu.VMEM((EB, 16), jnp.float32),
        pltpu.VMEM((EB, 16), jnp.float32),
        pltpu.VMEM((EB, 16), jnp.float32),
        pltpu.VMEM((EB, 16), jnp.float32),
        pltpu.SemaphoreType.DMA,
    ],
)
def _k2(ex_hbm, den0_hbm, den1_hbm, dst_hbm, w_hbm,
        didx, exv, d0v, d1v, wv, sem):
    cid = lax.axis_index("c")
    sid = lax.axis_index("s")
    wid = sid * NC + cid
    base0 = wid * PER_SUB
    row0_0 = wid * NBLK * 4

    def blk(b, _):
        base = base0 + b * EB
        row0 = row0_0 + b * 4
        pltpu.sync_copy(dst_hbm.at[pl.ds(_m4(row0), 4)], didx)
        cps = [pltpu.async_copy(ex_hbm.at[pl.ds(_m8(base), EB)], exv, sem)]
        for t in range(4):
            cps.append(pltpu.async_copy(
                den0_hbm.at[didx.at[t]], d0v.at[pl.ds(128 * t, 128)], sem))
            cps.append(pltpu.async_copy(
                den1_hbm.at[didx.at[t]], d1v.at[pl.ds(128 * t, 128)], sem))
        for cp in cps:
            cp.wait()

        def edge(i, _):
            wv[i, :] = exv[i, :] / (d0v[i, :] + d1v[i, :])
            return 0
        lax.fori_loop(0, EB, edge, 0)

        pltpu.sync_copy(wv, w_hbm.at[pl.ds(_m8(base), EB)])
        return 0
    lax.fori_loop(0, NBLK, blk, 0)


@functools.partial(
    pl.kernel, mesh=_MESH, compiler_params=_SC_PARAMS,
    out_type=[jax.ShapeDtypeStruct((8, NC, NP, 16), jnp.float32)],
    scratch_types=[
        pltpu.VMEM((NBLK * 4, 128), jnp.int32),
        pltpu.VMEM((NBLK * 4, 128), jnp.int32),
        pltpu.VMEM((PER_SUB,), jnp.float32),
        pltpu.VMEM((EB, 16), jnp.float32),
        pltpu.VMEM((EB, 16), jnp.float32),
        pltpu.VMEM_SHARED((NP, 16), jnp.float32),
        pltpu.SemaphoreType.DMA,
        pltpu.SemaphoreType.DMA,
    ],
)
def _k3(xc0, xc1, xc2, xc3, xc4, xc5, xc6, xc7, wt_hbm, src_hbm, dst_hbm,
        acc_hbm, sidx_all, didx_all, wrow, rows0, rows1, acc_sh, sem0, sem1):
    cid = lax.axis_index("c")
    sid = lax.axis_index("s")
    wid = sid * NC + cid
    zeros16 = jnp.zeros((16,), jnp.float32)
    base0 = wid * PER_SUB
    row0_0 = wid * NBLK * 4
    r0 = sid * ROWS_PER_SUB

    pltpu.sync_copy(src_hbm.at[pl.ds(_m8(row0_0), NBLK * 4)], sidx_all)
    pltpu.sync_copy(dst_hbm.at[pl.ds(_m8(row0_0), NBLK * 4)], didx_all)

    for c in range(8):
        xc = (xc0, xc1, xc2, xc3, xc4, xc5, xc6, xc7)[c]
        pltpu.sync_copy(wt_hbm.at[c, pl.ds(_m8(base0), PER_SUB)], wrow)

        def zb(i, _):
            rows0[i, :] = zeros16
            return 0
        lax.fori_loop(0, EB, zb, 0)
        for q in range(6):
            pltpu.sync_copy(
                rows0, acc_sh.at[pl.ds(_m8(r0 + q * EB), EB)])
        pltpu.sync_copy(rows0.at[pl.ds(0, ROWS_PER_SUB - 6 * EB)],
                        acc_sh.at[pl.ds(_m8(r0 + 6 * EB),
                                        ROWS_PER_SUB - 6 * EB)])
        plsc.subcore_barrier()

        def fire(b, buf, sem):
            for t in range(4):
                pltpu.async_copy(xc.at[sidx_all.at[b * 4 + t]],
                                 buf.at[pl.ds(128 * t, 128)], sem)

        def drain(buf, sem):
            for t in range(4):
                pltpu.make_async_copy(xc.at[pl.ds(0, 128)],
                                      buf.at[pl.ds(128 * t, 128)], sem).wait()

        def work(b, buf):
            def grp(g, _):
                wvec = wrow[pl.ds(b * EB + g * 16, 16)]
                for j in range(16):
                    i = g * 16 + j
                    gj = jnp.broadcast_to(wvec[j], (16,))
                    buf[i, :] = buf[i, :] * gj
                return 0
            lax.fori_loop(0, EB // 16, grp, 0)
            for t in range(4):
                pltpu.sync_copy(buf.at[pl.ds(128 * t, 128)],
                                acc_sh.at[didx_all.at[b * 4 + t]], add=True)

        fire(0, rows0, sem0)

        def pairs(p, _):
            b0 = 2 * p
            fire(b0 + 1, rows1, sem1)
            drain(rows0, sem0)
            work(b0, rows0)

            @pl.when(p < NBLK // 2 - 1)
            def _():
                fire(b0 + 2, rows0, sem0)
            drain(rows1, sem1)
            work(b0 + 1, rows1)
            return 0
        lax.fori_loop(0, NBLK // 2, pairs, 0)

        plsc.subcore_barrier()
        pltpu.sync_copy(acc_sh.at[pl.ds(_m8(r0), ROWS_PER_SUB)],
                        acc_hbm.at[c, cid, pl.ds(_m8(r0), ROWS_PER_SUB)])
        plsc.subcore_barrier()


def _wt_body(w_ref, out_ref):
    out_ref[...] = jnp.transpose(w_ref[...], (1, 0))[:8, :]


def _wt(w):
    BR = 4096
    nb = EP // BR
    return pl.pallas_call(
        _wt_body,
        grid=(nb,),
        in_specs=[pl.BlockSpec((BR, 16), lambda i: (i, 0))],
        out_specs=pl.BlockSpec((8, BR), lambda i: (0, i)),
        out_shape=jax.ShapeDtypeStruct((8, EP), jnp.float32),
    )(w)


def _final_body(acc_ref, wfc_ref, bfc_ref, out_ref):
    s = acc_ref[:, 0] + acc_ref[:, 1]
    feat = jnp.concatenate([s[c] for c in range(8)], axis=1)
    feat = jnp.maximum(feat, 0.0)
    out_ref[...] = jnp.dot(feat, wfc_ref[...],
                           preferred_element_type=jnp.float32) + bfc_ref[...]


def _final(acc, W_fc, b_fc):
    RB = 1000
    nb = N // RB
    return pl.pallas_call(
        _final_body,
        grid=(nb,),
        in_specs=[
            pl.BlockSpec((8, NC, RB, 16), lambda i: (0, 0, i, 0)),
            pl.BlockSpec((HID, OUT), lambda i: (0, 0)),
            pl.BlockSpec((1, OUT), lambda i: (0, 0)),
        ],
        out_specs=pl.BlockSpec((RB, OUT), lambda i: (i, 0)),
        out_shape=jax.ShapeDtypeStruct((N, OUT), jnp.float32),
    )(acc, W_fc, b_fc.reshape(1, OUT))


def _blockdiag(a):
    eye = jnp.eye(H, 16, dtype=jnp.float32)
    return (a[:, :, None] * eye[:, None, :]).reshape(HID, 16)


def _pad_edges(ei):
    src = jnp.pad(ei[0], (0, EP - E), constant_values=0)
    dst = jnp.pad(ei[1], (0, EP - E), constant_values=N)
    return (src.astype(jnp.int32).reshape(ER, 128),
            dst.astype(jnp.int32).reshape(ER, 128))


def _edge_type(als, ald_padded, xc, src, dst):
    ex, d0, d1 = _k1(als, ald_padded, src, dst)
    (w,) = _k2(ex, d0, d1, dst)
    wt = _wt(w)
    (acc,) = _k3(*xc, wt, src, dst)
    return acc


def kernel(x_reviewer, x_author, edge_index_r2a, edge_index_a2r,
           W_proj_rev, b_proj_rev, W_proj_aut, b_proj_aut,
           a_src_r2a, a_dst_r2a, a_src_a2r, a_dst_a2r,
           q_sem, W_k, b_k, W_fc, b_fc):
    xpR, alR_s_r2a, alR_d_a2r = _proj(
        x_reviewer, W_proj_rev, b_proj_rev,
        _blockdiag(a_src_r2a), _blockdiag(a_dst_a2r))
    xpA, alA_s_a2r, alA_d_r2a = _proj(
        x_author, W_proj_aut, b_proj_aut,
        _blockdiag(a_src_a2r), _blockdiag(a_dst_r2a))

    xcR = [xpR[:, 16 * c:16 * c + 16] for c in range(8)]
    xcA = [xpA[:, 16 * c:16 * c + 16] for c in range(8)]
    padN = ((0, NP - N), (0, 0))

    srcA, dstA = _pad_edges(edge_index_r2a)
    srcB, dstB = _pad_edges(edge_index_a2r)

    acc_aut = _edge_type(alR_s_r2a, jnp.pad(alA_d_r2a, padN), xcR, srcA, dstA)
    acc_rev = _edge_type(alA_s_a2r, jnp.pad(alR_d_a2r, padN), xcA, srcB, dstB)

    out_rev = _final(acc_rev, W_fc, b_fc)
    out_aut = _final(acc_aut, W_fc, b_fc)
    return jnp.concatenate([out_rev, out_aut], axis=0)


# speedup vs baseline: 10.5587x; 1.0468x over previous
import functools

import jax
import jax.numpy as jnp
from jax import lax
from jax.experimental import pallas as pl
from jax.experimental.pallas import tpu as pltpu
from jax.experimental.pallas import tpu_sc as plsc

N = 50000
NP = 50048
E = 500000
D_IN = 128
HID = 128
H = 8
DH = 16
OUT = 64
NC = 2
NS = 16
NW = NC * NS
EB = 512
NBLK = 32
PER_SUB = NBLK * EB
EP = PER_SUB * NW
ER = EP // 128
ROWS_PER_SUB = NP // NS


def _proj_body(x_ref, w_ref, b_ref, a1_ref, a2_ref, xp_ref, al1_ref, al2_ref):
    xp = jnp.dot(x_ref[...], w_ref[...], preferred_element_type=jnp.float32)
    xp = xp + b_ref[...]
    xp_ref[...] = xp
    al1_ref[...] = jnp.dot(xp, a1_ref[...], preferred_element_type=jnp.float32)
    al2_ref[...] = jnp.dot(xp, a2_ref[...], preferred_element_type=jnp.float32)


def _proj(x, W, b, A1, A2):
    RB = 1000
    nb = N // RB
    return pl.pallas_call(
        _proj_body,
        grid=(nb,),
        in_specs=[
            pl.BlockSpec((RB, D_IN), lambda i: (i, 0)),
            pl.BlockSpec((D_IN, HID), lambda i: (0, 0)),
            pl.BlockSpec((1, HID), lambda i: (0, 0)),
            pl.BlockSpec((HID, 16), lambda i: (0, 0)),
            pl.BlockSpec((HID, 16), lambda i: (0, 0)),
        ],
        out_specs=[
            pl.BlockSpec((RB, HID), lambda i: (i, 0)),
            pl.BlockSpec((RB, 16), lambda i: (i, 0)),
            pl.BlockSpec((RB, 16), lambda i: (i, 0)),
        ],
        out_shape=[
            jax.ShapeDtypeStruct((N, HID), jnp.float32),
            jax.ShapeDtypeStruct((N, 16), jnp.float32),
            jax.ShapeDtypeStruct((N, 16), jnp.float32),
        ],
    )(x, W, b.reshape(1, HID), A1, A2)


_MESH = plsc.VectorSubcoreMesh(core_axis_name="c", subcore_axis_name="s")
_SC_PARAMS = pltpu.CompilerParams(use_tc_tiling_on_sc=False)


def _m8(x):
    return pl.multiple_of(x, 8)


def _m4(x):
    return pl.multiple_of(x, 4)


@functools.partial(
    pl.kernel, mesh=_MESH, compiler_params=_SC_PARAMS,
    out_type=[
        jax.ShapeDtypeStruct((EP, 16), jnp.float32),
        jax.ShapeDtypeStruct((NP, 16), jnp.float32),
        jax.ShapeDtypeStruct((NP, 16), jnp.float32),
    ],
    scratch_types=[
        pltpu.VMEM((NBLK * 4, 128), jnp.int32),
        pltpu.VMEM((NBLK * 4, 128), jnp.int32),
        pltpu.VMEM((EB, 16), jnp.float32),
        pltpu.VMEM((EB, 16), jnp.float32),
        pltpu.VMEM((EB, 16), jnp.float32),
        pltpu.VMEM((EB, 16), jnp.float32),
        pltpu.VMEM((EB, 16), jnp.float32),
        pltpu.VMEM_SHARED((NP, 16), jnp.float32),
        pltpu.SemaphoreType.DMA,
        pltpu.SemaphoreType.DMA,
    ],
)
def _k1(als_hbm, ald_hbm, src_hbm, dst_hbm,
        ex_hbm, den0_hbm, den1_hbm,
        sidx_all, didx_all, as0, ad0, as1, ad1, exf, den_sh, sem0, sem1):
    cid = lax.axis_index("c")
    sid = lax.axis_index("s")
    wid = sid * NC + cid
    zeros16 = jnp.zeros((16,), jnp.float32)
    base0 = wid * PER_SUB
    row0_0 = wid * NBLK * 4
    r0 = sid * ROWS_PER_SUB

    pltpu.sync_copy(src_hbm.at[pl.ds(_m8(row0_0), NBLK * 4)], sidx_all)
    pltpu.sync_copy(dst_hbm.at[pl.ds(_m8(row0_0), NBLK * 4)], didx_all)

    if True:
        def zb(i, _):
            exf[i, :] = zeros16
            return 0
        lax.fori_loop(0, EB, zb, 0)
        for q in range(6):
            pltpu.sync_copy(exf, den_sh.at[pl.ds(_m8(r0 + q * EB), EB)])
        pltpu.sync_copy(exf.at[pl.ds(0, ROWS_PER_SUB - 6 * EB)],
                        den_sh.at[pl.ds(_m8(r0 + 6 * EB),
                                        ROWS_PER_SUB - 6 * EB)])
        plsc.subcore_barrier()

        def fire(b, asv, adv, sem):
            for t in range(4):
                pltpu.async_copy(als_hbm.at[sidx_all.at[b * 4 + t]],
                                 asv.at[pl.ds(128 * t, 128)], sem)
                pltpu.async_copy(ald_hbm.at[didx_all.at[b * 4 + t]],
                                 adv.at[pl.ds(128 * t, 128)], sem)

        def drain(asv, adv, sem):
            for t in range(4):
                pltpu.make_async_copy(als_hbm.at[pl.ds(0, 128)],
                                      asv.at[pl.ds(128 * t, 128)], sem).wait()
                pltpu.make_async_copy(ald_hbm.at[pl.ds(0, 128)],
                                      adv.at[pl.ds(128 * t, 128)], sem).wait()

        def work(b, asv, adv):
            def edge(i, _):
                s = asv[i, :] + adv[i, :]
                s = jnp.maximum(s, 0.0) + 0.2 * jnp.minimum(s, 0.0)
                exf[i, :] = jnp.exp(s)
                return 0
            lax.fori_loop(0, EB, edge, 0)
            base = base0 + b * EB
            pltpu.sync_copy(exf, ex_hbm.at[pl.ds(_m8(base), EB)])
            for t in range(4):
                pltpu.sync_copy(exf.at[pl.ds(128 * t, 128)],
                                den_sh.at[didx_all.at[b * 4 + t]], add=True)

        fire(0, as0, ad0, sem0)

        def pairs(p, _):
            b0 = 2 * p
            fire(b0 + 1, as1, ad1, sem1)
            drain(as0, ad0, sem0)
            work(b0, as0, ad0)

            @pl.when(p < NBLK // 2 - 1)
            def _():
                fire(b0 + 2, as0, ad0, sem0)
            drain(as1, ad1, sem1)
            work(b0 + 1, as1, ad1)
            return 0
        lax.fori_loop(0, NBLK // 2, pairs, 0)

        plsc.subcore_barrier()

        @pl.when(cid == 0)
        def _():
            pltpu.sync_copy(den_sh.at[pl.ds(_m8(r0), ROWS_PER_SUB)],
                            den0_hbm.at[pl.ds(_m8(r0), ROWS_PER_SUB)])

        @pl.when(cid == 1)
        def _():
            pltpu.sync_copy(den_sh.at[pl.ds(_m8(r0), ROWS_PER_SUB)],
                            den1_hbm.at[pl.ds(_m8(r0), ROWS_PER_SUB)])


@functools.partial(
    pl.kernel, mesh=_MESH, compiler_params=_SC_PARAMS,
    out_type=[jax.ShapeDtypeStruct((EP, 16), jnp.float32)],
    scratch_types=[
        pltpu.VMEM((NBLK * 4, 128), jnp.int32),
        pltpu.VMEM((EB, 16), jnp.float32),
        pltpu.VMEM((EB, 16), jnp.float32),
        pltpu.VMEM((EB, 16), jnp.float32),
        pltpu.VMEM((EB, 16), jnp.float32),
        pltpu.VMEM((EB, 16), jnp.float32),
        pltpu.VMEM((EB, 16), jnp.float32),
        pltpu.VMEM((EB, 16), jnp.float32),
        pltpu.SemaphoreType.DMA,
        pltpu.SemaphoreType.DMA,
    ],
)
def _k2(ex_hbm, den0_hbm, den1_hbm, dst_hbm, w_hbm,
        didx_all, ex0, d00, d10, ex1, d01, d11, wv, sem0, sem1):
    cid = lax.axis_index("c")
    sid = lax.axis_index("s")
    wid = sid * NC + cid
    base0 = wid * PER_SUB
    row0_0 = wid * NBLK * 4

    pltpu.sync_copy(dst_hbm.at[pl.ds(_m8(row0_0), NBLK * 4)], didx_all)

    def fire(b, exv, d0v, d1v, sem):
        base = base0 + b * EB
        pltpu.async_copy(ex_hbm.at[pl.ds(_m8(base), EB)], exv, sem)
        for t in range(4):
            pltpu.async_copy(den0_hbm.at[didx_all.at[b * 4 + t]],
                             d0v.at[pl.ds(128 * t, 128)], sem)
            pltpu.async_copy(den1_hbm.at[didx_all.at[b * 4 + t]],
                             d1v.at[pl.ds(128 * t, 128)], sem)

    def drain(exv, d0v, d1v, sem):
        pltpu.make_async_copy(ex_hbm.at[pl.ds(0, EB)], exv, sem).wait()
        for t in range(4):
            pltpu.make_async_copy(den0_hbm.at[pl.ds(0, 128)],
                                  d0v.at[pl.ds(128 * t, 128)], sem).wait()
            pltpu.make_async_copy(den1_hbm.at[pl.ds(0, 128)],
                                  d1v.at[pl.ds(128 * t, 128)], sem).wait()

    def work(b, exv, d0v, d1v):
        def edge(i, _):
            wv[i, :] = exv[i, :] / (d0v[i, :] + d1v[i, :])
            return 0
        lax.fori_loop(0, EB, edge, 0)
        base = base0 + b * EB
        pltpu.sync_copy(wv, w_hbm.at[pl.ds(_m8(base), EB)])

    fire(0, ex0, d00, d10, sem0)

    def pairs(p, _):
        b0 = 2 * p
        fire(b0 + 1, ex1, d01, d11, sem1)
        drain(ex0, d00, d10, sem0)
        work(b0, ex0, d00, d10)

        @pl.when(p < NBLK // 2 - 1)
        def _():
            fire(b0 + 2, ex0, d00, d10, sem0)
        drain(ex1, d01, d11, sem1)
        work(b0 + 1, ex1, d01, d11)
        return 0
    lax.fori_loop(0, NBLK // 2, pairs, 0)


@functools.partial(
    pl.kernel, mesh=_MESH, compiler_params=_SC_PARAMS,
    out_type=[jax.ShapeDtypeStruct((8, NC, NP, 16), jnp.float32)],
    scratch_types=[
        pltpu.VMEM((NBLK * 4, 128), jnp.int32),
        pltpu.VMEM((NBLK * 4, 128), jnp.int32),
        pltpu.VMEM((PER_SUB,), jnp.float32),
        pltpu.VMEM((EB, 16), jnp.float32),
        pltpu.VMEM((EB, 16), jnp.float32),
        pltpu.VMEM_SHARED((NP, 16), jnp.float32),
        pltpu.SemaphoreType.DMA,
        pltpu.SemaphoreType.DMA,
    ],
)
def _k3(xc0, xc1, xc2, xc3, xc4, xc5, xc6, xc7, wt_hbm, src_hbm, dst_hbm,
        acc_hbm, sidx_all, didx_all, wrow, rows0, rows1, acc_sh, sem0, sem1):
    cid = lax.axis_index("c")
    sid = lax.axis_index("s")
    wid = sid * NC + cid
    zeros16 = jnp.zeros((16,), jnp.float32)
    base0 = wid * PER_SUB
    row0_0 = wid * NBLK * 4
    r0 = sid * ROWS_PER_SUB

    pltpu.sync_copy(src_hbm.at[pl.ds(_m8(row0_0), NBLK * 4)], sidx_all)
    pltpu.sync_copy(dst_hbm.at[pl.ds(_m8(row0_0), NBLK * 4)], didx_all)

    for c in range(8):
        xc = (xc0, xc1, xc2, xc3, xc4, xc5, xc6, xc7)[c]
        pltpu.sync_copy(wt_hbm.at[c, pl.ds(_m8(base0), PER_SUB)], wrow)

        def zb(i, _):
            rows0[i, :] = zeros16
            return 0
        lax.fori_loop(0, EB, zb, 0)
        for q in range(6):
            pltpu.sync_copy(
                rows0, acc_sh.at[pl.ds(_m8(r0 + q * EB), EB)])
        pltpu.sync_copy(rows0.at[pl.ds(0, ROWS_PER_SUB - 6 * EB)],
                        acc_sh.at[pl.ds(_m8(r0 + 6 * EB),
                                        ROWS_PER_SUB - 6 * EB)])
        plsc.subcore_barrier()

        def fire(b, buf, sem):
            for t in range(4):
                pltpu.async_copy(xc.at[sidx_all.at[b * 4 + t]],
                                 buf.at[pl.ds(128 * t, 128)], sem)

        def drain(buf, sem):
            for t in range(4):
                pltpu.make_async_copy(xc.at[pl.ds(0, 128)],
                                      buf.at[pl.ds(128 * t, 128)], sem).wait()

        def work(b, buf):
            def grp(g, _):
                wvec = wrow[pl.ds(b * EB + g * 16, 16)]
                for j in range(16):
                    i = g * 16 + j
                    gj = jnp.broadcast_to(wvec[j], (16,))
                    buf[i, :] = buf[i, :] * gj
                return 0
            lax.fori_loop(0, EB // 16, grp, 0)
            for t in range(4):
                pltpu.sync_copy(buf.at[pl.ds(128 * t, 128)],
                                acc_sh.at[didx_all.at[b * 4 + t]], add=True)

        fire(0, rows0, sem0)

        def pairs(p, _):
            b0 = 2 * p
            fire(b0 + 1, rows1, sem1)
            drain(rows0, sem0)
            work(b0, rows0)

            @pl.when(p < NBLK // 2 - 1)
            def _():
                fire(b0 + 2, rows0, sem0)
            drain(rows1, sem1)
            work(b0 + 1, rows1)
            return 0
        lax.fori_loop(0, NBLK // 2, pairs, 0)

        plsc.subcore_barrier()
        pltpu.sync_copy(acc_sh.at[pl.ds(_m8(r0), ROWS_PER_SUB)],
                        acc_hbm.at[c, cid, pl.ds(_m8(r0), ROWS_PER_SUB)])
        plsc.subcore_barrier()


def _wt_body(w_ref, out_ref):
    out_ref[...] = jnp.transpose(w_ref[...], (1, 0))[:8, :]


def _wt(w):
    BR = 4096
    nb = EP // BR
    return pl.pallas_call(
        _wt_body,
        grid=(nb,),
        in_specs=[pl.BlockSpec((BR, 16), lambda i: (i, 0))],
        out_specs=pl.BlockSpec((8, BR), lambda i: (0, i)),
        out_shape=jax.ShapeDtypeStruct((8, EP), jnp.float32),
    )(w)


def _final_body(acc_ref, wfc_ref, bfc_ref, out_ref):
    s = acc_ref[:, 0] + acc_ref[:, 1]
    feat = jnp.concatenate([s[c] for c in range(8)], axis=1)
    feat = jnp.maximum(feat, 0.0)
    out_ref[...] = jnp.dot(feat, wfc_ref[...],
                           preferred_element_type=jnp.float32) + bfc_ref[...]


def _final(acc, W_fc, b_fc):
    RB = 1000
    nb = N // RB
    return pl.pallas_call(
        _final_body,
        grid=(nb,),
        in_specs=[
            pl.BlockSpec((8, NC, RB, 16), lambda i: (0, 0, i, 0)),
            pl.BlockSpec((HID, OUT), lambda i: (0, 0)),
            pl.BlockSpec((1, OUT), lambda i: (0, 0)),
        ],
        out_specs=pl.BlockSpec((RB, OUT), lambda i: (i, 0)),
        out_shape=jax.ShapeDtypeStruct((N, OUT), jnp.float32),
    )(acc, W_fc, b_fc.reshape(1, OUT))


def _blockdiag(a):
    eye = jnp.eye(H, 16, dtype=jnp.float32)
    return (a[:, :, None] * eye[:, None, :]).reshape(HID, 16)


def _pad_edges(ei):
    src = jnp.pad(ei[0], (0, EP - E), constant_values=0)
    dst = jnp.pad(ei[1], (0, EP - E), constant_values=N)
    return (src.astype(jnp.int32).reshape(ER, 128),
            dst.astype(jnp.int32).reshape(ER, 128))


def _edge_type(als, ald_padded, xc, src, dst):
    ex, d0, d1 = _k1(als, ald_padded, src, dst)
    (w,) = _k2(ex, d0, d1, dst)
    wt = _wt(w)
    (acc,) = _k3(*xc, wt, src, dst)
    return acc


def kernel(x_reviewer, x_author, edge_index_r2a, edge_index_a2r,
           W_proj_rev, b_proj_rev, W_proj_aut, b_proj_aut,
           a_src_r2a, a_dst_r2a, a_src_a2r, a_dst_a2r,
           q_sem, W_k, b_k, W_fc, b_fc):
    xpR, alR_s_r2a, alR_d_a2r = _proj(
        x_reviewer, W_proj_rev, b_proj_rev,
        _blockdiag(a_src_r2a), _blockdiag(a_dst_a2r))
    xpA, alA_s_a2r, alA_d_r2a = _proj(
        x_author, W_proj_aut, b_proj_aut,
        _blockdiag(a_src_a2r), _blockdiag(a_dst_r2a))

    xcR = [xpR[:, 16 * c:16 * c + 16] for c in range(8)]
    xcA = [xpA[:, 16 * c:16 * c + 16] for c in range(8)]
    padN = ((0, NP - N), (0, 0))

    srcA, dstA = _pad_edges(edge_index_r2a)
    srcB, dstB = _pad_edges(edge_index_a2r)

    acc_aut = _edge_type(alR_s_r2a, jnp.pad(alA_d_r2a, padN), xcR, srcA, dstA)
    acc_rev = _edge_type(alA_s_a2r, jnp.pad(alR_d_a2r, padN), xcA, srcB, dstB)

    out_rev = _final(acc_rev, W_fc, b_fc)
    out_aut = _final(acc_aut, W_fc, b_fc)
    return jnp.concatenate([out_rev, out_aut], axis=0)
